# baseline (speedup 1.0000x reference)
import jax
import jax.numpy as jnp
from jax import lax
from jax.experimental import pallas as pl
from jax.experimental.pallas import tpu as pltpu
from jax.experimental.pallas import tpu_sc as plsc

N = 100000
E = 1600000
D = 32
H = 16
EPS = 1e-3
L = 16

NUM_TILES = 16
G = 128
ROWS_TOTAL = E // G
ROWS_BASE = ROWS_TOTAL // NUM_TILES
ROWS_XTRA = ROWS_TOTAL % NUM_TILES
KA = 8
KB = 4
KS = KA + KB
STEPS = ROWS_BASE // KS
N_ACC = 100096
ROWS_ACC_TILE = N_ACC // NUM_TILES
LAST_TILE_OUT = N - (NUM_TILES - 1) * ROWS_ACC_TILE


def _segsum_body(edge_hbm, tab_hbm, out_hbm,
                 sflatA, dflatA, s2dA, d2dA, rowsA,
                 sflatB, dflatB, s2dB, d2dB, rowsB,
                 acc, gsemA, ssemA, isemA, gsemB, ssemB, isemB):
  cid = lax.axis_index("c")
  sid = lax.axis_index("s")

  def _zero_buf(j, _):
    rowsA[0, j, :] = jnp.zeros((H,), jnp.float32)
    return 0
  lax.fori_loop(0, G, _zero_buf, 0)
  zbuf = rowsA.at[0]
  acc_base = sid * ROWS_ACC_TILE
  for r in range(0, ROWS_ACC_TILE - G + 1, G):
    pltpu.sync_copy(zbuf, acc.at[pl.ds(acc_base + r, G)])
  rem = ROWS_ACC_TILE % G
  if rem:
    pltpu.sync_copy(zbuf.at[pl.ds(0, rem)],
                    acc.at[pl.ds(acc_base + ROWS_ACC_TILE - rem, rem)])
  plsc.subcore_barrier()

  tile_row0 = sid * ROWS_BASE + jnp.minimum(sid, ROWS_XTRA)
  n_rows = ROWS_BASE + jnp.where(sid < ROWS_XTRA, 1, 0)

  def _idx_start(row0, k, sflat, dflat, isem):
    e0 = row0 * G
    pltpu.async_copy(edge_hbm.at[0, pl.ds(e0, k * G)],
                     sflat.at[pl.ds(0, k * G)], isem)
    pltpu.async_copy(edge_hbm.at[1, pl.ds(e0, k * G)],
                     dflat.at[pl.ds(0, k * G)], isem)

  def _idx_wait(row0, k, sflat, dflat, isem):
    e0 = row0 * G
    pltpu.make_async_copy(edge_hbm.at[0, pl.ds(e0, k * G)],
                          sflat.at[pl.ds(0, k * G)], isem).wait()
    pltpu.make_async_copy(edge_hbm.at[1, pl.ds(e0, k * G)],
                          dflat.at[pl.ds(0, k * G)], isem).wait()

  def _transform_gather(k, sflat, dflat, s2d, d2d, rows, gsem):
    for i in range(k * G // L):
      j, l = divmod(i, G // L)
      v = sflat[pl.ds(i * L, L)]
      s2d[j, pl.ds(l * L, L)] = v + v + cid
      d2d[j, pl.ds(l * L, L)] = dflat[pl.ds(i * L, L)]
    for j in range(k):
      pltpu.async_copy(tab_hbm.at[s2d.at[j]], rows.at[j], gsem)

  def _gather_wait(k, s2d, rows, gsem):
    for j in range(k):
      pltpu.make_async_copy(tab_hbm.at[s2d.at[j]], rows.at[j], gsem).wait()

  def _scatter_issue(k, d2d, rows, ssem):
    for j in range(k):
      pltpu.async_copy(rows.at[j], acc.at[d2d.at[j]], ssem, add=True)

  def _scatter_wait(k, d2d, rows, ssem):
    for j in range(k):
      pltpu.make_async_copy(rows.at[j], acc.at[d2d.at[j]], ssem).wait()

  def _step(t, _):
    row0 = tile_row0 + t * KS

    @pl.when(t < STEPS - 1)
    def _():
      _idx_start(row0 + KS, KA, sflatA, dflatA, isemA)
    _gather_wait(KA, s2dA, rowsA, gsemA)
    _scatter_issue(KA, d2dA, rowsA, ssemA)

    _idx_wait(row0 + KA, KB, sflatB, dflatB, isemB)

    @pl.when(t > 0)
    def _():
      _scatter_wait(KB, d2dB, rowsB, ssemB)
    _transform_gather(KB, sflatB, dflatB, s2dB, d2dB, rowsB, gsemB)
    _gather_wait(KB, s2dB, rowsB, gsemB)
    _scatter_issue(KB, d2dB, rowsB, ssemB)

    _scatter_wait(KA, d2dA, rowsA, ssemA)

    @pl.when(t < STEPS - 1)
    def _():
      _idx_wait(row0 + KS, KA, sflatA, dflatA, isemA)
      _transform_gather(KA, sflatA, dflatA, s2dA, d2dA, rowsA, gsemA)
      _idx_start(row0 + KS + KA, KB, sflatB, dflatB, isemB)
    return 0

  _idx_start(tile_row0, KA, sflatA, dflatA, isemA)
  _idx_wait(tile_row0, KA, sflatA, dflatA, isemA)
  _transform_gather(KA, sflatA, dflatA, s2dA, d2dA, rowsA, gsemA)
  _idx_start(tile_row0 + KA, KB, sflatB, dflatB, isemB)
  lax.fori_loop(0, STEPS, _step, 0)
  _scatter_wait(KB, d2dB, rowsB, ssemB)

  def _tail(t, _):
    row0 = tile_row0 + STEPS * KS + t
    _idx_start(row0, 1, sflatB, dflatB, isemB)
    _idx_wait(row0, 1, sflatB, dflatB, isemB)
    _transform_gather(1, sflatB, dflatB, s2dB, d2dB, rowsB, gsemB)
    _gather_wait(1, s2dB, rowsB, gsemB)
    _scatter_issue(1, d2dB, rowsB, ssemB)
    _scatter_wait(1, d2dB, rowsB, ssemB)
    return 0
  lax.fori_loop(0, n_rows - STEPS * KS, _tail, 0)

  plsc.subcore_barrier()

  @pl.when(sid < NUM_TILES - 1)
  def _():
    pltpu.sync_copy(acc.at[pl.ds(acc_base, ROWS_ACC_TILE)],
                    out_hbm.at[cid, pl.ds(acc_base, ROWS_ACC_TILE)])

  @pl.when(sid == NUM_TILES - 1)
  def _():
    pltpu.sync_copy(acc.at[pl.ds(acc_base, LAST_TILE_OUT)],
                    out_hbm.at[cid, pl.ds(acc_base, LAST_TILE_OUT)])


def _segsum(edge_index, tab):
  mesh = plsc.VectorSubcoreMesh(core_axis_name="c", subcore_axis_name="s")
  f = pl.kernel(
      _segsum_body,
      out_type=jax.ShapeDtypeStruct((2, N, H), jnp.float32),
      mesh=mesh,
      scratch_types=[
          pltpu.VMEM((KA * G,), jnp.int32),
          pltpu.VMEM((KA * G,), jnp.int32),
          pltpu.VMEM((KA, G), jnp.int32),
          pltpu.VMEM((KA, G), jnp.int32),
          pltpu.VMEM((KA, G, H), jnp.float32),
          pltpu.VMEM((KB * G,), jnp.int32),
          pltpu.VMEM((KB * G,), jnp.int32),
          pltpu.VMEM((KB, G), jnp.int32),
          pltpu.VMEM((KB, G), jnp.int32),
          pltpu.VMEM((KB, G, H), jnp.float32),
          pltpu.VMEM_SHARED((N_ACC, H), jnp.float32),
          pltpu.SemaphoreType.DMA,
          pltpu.SemaphoreType.DMA,
          pltpu.SemaphoreType.DMA,
          pltpu.SemaphoreType.DMA,
          pltpu.SemaphoreType.DMA,
          pltpu.SemaphoreType.DMA,
      ],
      compiler_params=pltpu.CompilerParams(use_tc_tiling_on_sc=False),
  )
  return f(edge_index, tab)


def _mm1_body(s_ref, w_ref, b_ref, o_ref):
  sa = s_ref[0]
  sb = s_ref[1]
  h = jnp.dot(sa, w_ref[:H, :], preferred_element_type=jnp.float32)
  h += jnp.dot(sb, w_ref[H:, :], preferred_element_type=jnp.float32)
  o_ref[...] = jnp.maximum(h + b_ref[0], 0.0)


def _mm2_body(s_ref, x_ref, w_ref, b_ref, o_ref):
  sa = s_ref[0]
  sb = s_ref[1]
  h = jnp.dot(sa, w_ref[:H, :], preferred_element_type=jnp.float32)
  h += jnp.dot(sb, w_ref[H:, :], preferred_element_type=jnp.float32)
  o_ref[...] = jnp.maximum(h + b_ref[0] + x_ref[...], 0.0)


_BR = 4000


def _mm1(s, w, b):
  return pl.pallas_call(
      _mm1_body,
      grid=(N // _BR,),
      in_specs=[
          pl.BlockSpec((2, _BR, H), lambda i: (0, i, 0)),
          pl.BlockSpec((D, D), lambda i: (0, 0)),
          pl.BlockSpec((1, D), lambda i: (0, 0)),
      ],
      out_specs=pl.BlockSpec((_BR, D), lambda i: (i, 0)),
      out_shape=jax.ShapeDtypeStruct((N, D), jnp.float32),
  )(s, w, b)


def _mm2(s, x, w, b):
  return pl.pallas_call(
      _mm2_body,
      grid=(N // _BR,),
      in_specs=[
          pl.BlockSpec((2, _BR, H), lambda i: (0, i, 0)),
          pl.BlockSpec((_BR, D), lambda i: (i, 0)),
          pl.BlockSpec((D, D), lambda i: (0, 0)),
          pl.BlockSpec((1, D), lambda i: (0, 0)),
      ],
      out_specs=pl.BlockSpec((_BR, D), lambda i: (i, 0)),
      out_shape=jax.ShapeDtypeStruct((N, D), jnp.float32),
  )(s, x, w, b)


def kernel(x, edge_index, W1, W2, g1, b1, m1, v1, g2, b2, m2, v2):
  s1 = g1 / jnp.sqrt(v1 + EPS)
  w1f = W1 * s1[None, :]
  b1f = (b1 - m1 * s1)[None, :]
  s2 = g2 / jnp.sqrt(v2 + EPS)
  w2f = W2 * s2[None, :]
  b2f = (b2 - m2 * s2)[None, :]

  s_1 = _segsum(edge_index, x.reshape(2 * N, H))
  h1 = _mm1(s_1, w1f, b1f)
  s_2 = _segsum(edge_index, h1.reshape(2 * N, H))
  return _mm2(s_2, x, w2f, b2f)


# speedup vs baseline: 1.3545x; 1.3545x over previous
import jax
import jax.numpy as jnp
from jax import lax
from jax.experimental import pallas as pl
from jax.experimental.pallas import tpu as pltpu
from jax.experimental.pallas import tpu_sc as plsc

N = 100000
E = 1600000
D = 32
H = 16
EPS = 1e-3
L = 16

NUM_TILES = 16
G = 128
ROWS_TOTAL = E // G
ROWS_BASE = ROWS_TOTAL // NUM_TILES
ROWS_XTRA = ROWS_TOTAL % NUM_TILES
KA = 8
KB = 4
KS = KA + KB
STEPS = ROWS_BASE // KS
N_ACC = 100096
ROWS_ACC_TILE = N_ACC // NUM_TILES


def _segsum_body(edge_hbm, tab_hbm, out_hbm,
                 sflatA, dflatA, s2dA, d2dA, rowsA,
                 sflatB, dflatB, s2dB, d2dB, rowsB,
                 acc, gsemA, ssemA, isemA, gsemB, ssemB, isemB):
  cid = lax.axis_index("c")
  sid = lax.axis_index("s")

  def _zero_buf(j, _):
    rowsA[0, j, :] = jnp.zeros((H,), jnp.float32)
    return 0
  lax.fori_loop(0, G, _zero_buf, 0)
  zbuf = rowsA.at[0]
  acc_base = sid * ROWS_ACC_TILE
  for r in range(0, ROWS_ACC_TILE - G + 1, G):
    pltpu.sync_copy(zbuf, acc.at[pl.ds(acc_base + r, G)])
  rem = ROWS_ACC_TILE % G
  if rem:
    pltpu.sync_copy(zbuf.at[pl.ds(0, rem)],
                    acc.at[pl.ds(acc_base + ROWS_ACC_TILE - rem, rem)])
  plsc.subcore_barrier()

  tile_row0 = sid * ROWS_BASE + jnp.minimum(sid, ROWS_XTRA)
  n_rows = ROWS_BASE + jnp.where(sid < ROWS_XTRA, 1, 0)

  def _idx_start(row0, k, sflat, dflat, isem):
    e0 = row0 * G
    pltpu.async_copy(edge_hbm.at[0, pl.ds(e0, k * G)],
                     sflat.at[pl.ds(0, k * G)], isem)
    pltpu.async_copy(edge_hbm.at[1, pl.ds(e0, k * G)],
                     dflat.at[pl.ds(0, k * G)], isem)

  def _idx_wait(row0, k, sflat, dflat, isem):
    e0 = row0 * G
    pltpu.make_async_copy(edge_hbm.at[0, pl.ds(e0, k * G)],
                          sflat.at[pl.ds(0, k * G)], isem).wait()
    pltpu.make_async_copy(edge_hbm.at[1, pl.ds(e0, k * G)],
                          dflat.at[pl.ds(0, k * G)], isem).wait()

  def _transform_gather(k, sflat, dflat, s2d, d2d, rows, gsem):
    for i in range(k * G // L):
      j, l = divmod(i, G // L)
      v = sflat[pl.ds(i * L, L)]
      s2d[j, pl.ds(l * L, L)] = v + v + cid
      d2d[j, pl.ds(l * L, L)] = dflat[pl.ds(i * L, L)]
    for j in range(k):
      pltpu.async_copy(tab_hbm.at[s2d.at[j]], rows.at[j], gsem)

  def _gather_wait(k, s2d, rows, gsem):
    for j in range(k):
      pltpu.make_async_copy(tab_hbm.at[s2d.at[j]], rows.at[j], gsem).wait()

  def _scatter_issue(k, d2d, rows, ssem):
    for j in range(k):
      pltpu.async_copy(rows.at[j], acc.at[d2d.at[j]], ssem, add=True)

  def _scatter_wait(k, d2d, rows, ssem):
    for j in range(k):
      pltpu.make_async_copy(rows.at[j], acc.at[d2d.at[j]], ssem).wait()

  def _step(t, _):
    row0 = tile_row0 + t * KS

    @pl.when(t < STEPS - 1)
    def _():
      _idx_start(row0 + KS, KA, sflatA, dflatA, isemA)
    _gather_wait(KA, s2dA, rowsA, gsemA)
    _scatter_issue(KA, d2dA, rowsA, ssemA)

    _idx_wait(row0 + KA, KB, sflatB, dflatB, isemB)

    @pl.when(t > 0)
    def _():
      _scatter_wait(KB, d2dB, rowsB, ssemB)
    _transform_gather(KB, sflatB, dflatB, s2dB, d2dB, rowsB, gsemB)
    _gather_wait(KB, s2dB, rowsB, gsemB)
    _scatter_issue(KB, d2dB, rowsB, ssemB)

    _scatter_wait(KA, d2dA, rowsA, ssemA)

    @pl.when(t < STEPS - 1)
    def _():
      _idx_wait(row0 + KS, KA, sflatA, dflatA, isemA)
      _transform_gather(KA, sflatA, dflatA, s2dA, d2dA, rowsA, gsemA)
      _idx_start(row0 + KS + KA, KB, sflatB, dflatB, isemB)
    return 0

  _idx_start(tile_row0, KA, sflatA, dflatA, isemA)
  _idx_wait(tile_row0, KA, sflatA, dflatA, isemA)
  _transform_gather(KA, sflatA, dflatA, s2dA, d2dA, rowsA, gsemA)
  _idx_start(tile_row0 + KA, KB, sflatB, dflatB, isemB)
  lax.fori_loop(0, STEPS, _step, 0)
  _scatter_wait(KB, d2dB, rowsB, ssemB)

  def _tail(t, _):
    row0 = tile_row0 + STEPS * KS + t
    _idx_start(row0, 1, sflatB, dflatB, isemB)
    _idx_wait(row0, 1, sflatB, dflatB, isemB)
    _transform_gather(1, sflatB, dflatB, s2dB, d2dB, rowsB, gsemB)
    _gather_wait(1, s2dB, rowsB, gsemB)
    _scatter_issue(1, d2dB, rowsB, ssemB)
    _scatter_wait(1, d2dB, rowsB, ssemB)
    return 0
  lax.fori_loop(0, n_rows - STEPS * KS, _tail, 0)

  plsc.subcore_barrier()

  pltpu.sync_copy(acc.at[pl.ds(acc_base, ROWS_ACC_TILE)],
                  out_hbm.at[cid, pl.ds(acc_base, ROWS_ACC_TILE)])


def _segsum(edge_index, tab):
  mesh = plsc.VectorSubcoreMesh(core_axis_name="c", subcore_axis_name="s")
  f = pl.kernel(
      _segsum_body,
      out_type=jax.ShapeDtypeStruct((2, N_ACC, H), jnp.float32),
      mesh=mesh,
      scratch_types=[
          pltpu.VMEM((KA * G,), jnp.int32),
          pltpu.VMEM((KA * G,), jnp.int32),
          pltpu.VMEM((KA, G), jnp.int32),
          pltpu.VMEM((KA, G), jnp.int32),
          pltpu.VMEM((KA, G, H), jnp.float32),
          pltpu.VMEM((KB * G,), jnp.int32),
          pltpu.VMEM((KB * G,), jnp.int32),
          pltpu.VMEM((KB, G), jnp.int32),
          pltpu.VMEM((KB, G), jnp.int32),
          pltpu.VMEM((KB, G, H), jnp.float32),
          pltpu.VMEM_SHARED((N_ACC, H), jnp.float32),
          pltpu.SemaphoreType.DMA,
          pltpu.SemaphoreType.DMA,
          pltpu.SemaphoreType.DMA,
          pltpu.SemaphoreType.DMA,
          pltpu.SemaphoreType.DMA,
          pltpu.SemaphoreType.DMA,
      ],
      compiler_params=pltpu.CompilerParams(use_tc_tiling_on_sc=False),
  )
  return f(edge_index, tab)


def _mm1_body(s_ref, w0_ref, w1_ref, b_ref, o_ref):
  h = jnp.dot(s_ref[0], w0_ref[...], preferred_element_type=jnp.float32)
  h += jnp.dot(s_ref[1], w1_ref[...], preferred_element_type=jnp.float32)
  h = jnp.maximum(h + b_ref[0], 0.0)
  o_ref[:, 0, :] = h[:, :G]
  o_ref[:, 1, :] = h[:, G:]


def _mm2_body(s_ref, x_ref, w0_ref, w1_ref, b_ref, o_ref):
  h = jnp.dot(s_ref[0], w0_ref[...], preferred_element_type=jnp.float32)
  h += jnp.dot(s_ref[1], w1_ref[...], preferred_element_type=jnp.float32)
  h += b_ref[0]
  o_ref[:, 0, :] = jnp.maximum(h[:, :G] + x_ref[:, 0, :], 0.0)
  o_ref[:, 1, :] = jnp.maximum(h[:, G:] + x_ref[:, 1, :], 0.0)


_BR = 3128
_NP = N * D // G // 2
_NPF = N_ACC * D // G // 2


def _mm1(s, w0, w1, b):
  return pl.pallas_call(
      _mm1_body,
      grid=(_NPF // _BR,),
      in_specs=[
          pl.BlockSpec((2, _BR, G), lambda i: (0, i, 0)),
          pl.BlockSpec((G, 2 * G), lambda i: (0, 0)),
          pl.BlockSpec((G, 2 * G), lambda i: (0, 0)),
          pl.BlockSpec((1, 2 * G), lambda i: (0, 0)),
      ],
      out_specs=pl.BlockSpec((_BR, 2, G), lambda i: (i, 0, 0)),
      out_shape=jax.ShapeDtypeStruct((_NPF, 2, G), jnp.float32),
  )(s, w0, w1, b)


def _mm2(s, xp, w0, w1, b):
  return pl.pallas_call(
      _mm2_body,
      grid=(_NPF // _BR,),
      in_specs=[
          pl.BlockSpec((2, _BR, G), lambda i: (0, i, 0)),
          pl.BlockSpec((_BR, 2, G), lambda i: (i, 0, 0)),
          pl.BlockSpec((G, 2 * G), lambda i: (0, 0)),
          pl.BlockSpec((G, 2 * G), lambda i: (0, 0)),
          pl.BlockSpec((1, 2 * G), lambda i: (0, 0)),
      ],
      out_specs=pl.BlockSpec((_BR, 2, G), lambda i: (i, 0, 0)),
      out_shape=jax.ShapeDtypeStruct((_NP, 2, G), jnp.float32),
  )(s, xp, w0, w1, b)


def _pack_weights(wf):
  eye8 = jnp.eye(8, dtype=jnp.float32)
  ea, eb = eye8[:, :4], eye8[:, 4:]
  w0 = jnp.concatenate(
      [jnp.kron(ea, wf[:H, :]), jnp.kron(eb, wf[:H, :])], axis=1)
  w1 = jnp.concatenate(
      [jnp.kron(ea, wf[H:, :]), jnp.kron(eb, wf[H:, :])], axis=1)
  return w0, w1


def kernel(x, edge_index, W1, W2, g1, b1, m1, v1, g2, b2, m2, v2):
  s1 = g1 / jnp.sqrt(v1 + EPS)
  w1f = W1 * s1[None, :]
  b1f = (b1 - m1 * s1)[None, :]
  s2 = g2 / jnp.sqrt(v2 + EPS)
  w2f = W2 * s2[None, :]
  b2f = (b2 - m2 * s2)[None, :]

  s_1 = _segsum(edge_index, x.reshape(2 * N, H))
  s_1 = s_1.reshape(2, _NPF, G)
  w10, w11 = _pack_weights(w1f)
  bp1 = jnp.tile(b1f.reshape(-1), 8)[None]
  h1 = _mm1(s_1, w10, w11, bp1)
  s_2 = _segsum(edge_index, h1.reshape(2 * N_ACC, H))
  s_2 = s_2.reshape(2, _NPF, G)
  w20, w21 = _pack_weights(w2f)
  bp2 = jnp.tile(b2f.reshape(-1), 8)[None]
  out = _mm2(s_2, x.reshape(_NP, 2, G), w20, w21, bp2)
  return out.reshape(N, D)
